# baseline (speedup 1.0000x reference)
import functools

import jax
import jax.numpy as jnp
from jax import lax
from jax.experimental import pallas as pl
from jax.experimental.pallas import tpu as pltpu
from jax.experimental.pallas import tpu_sc as plsc

N = 10000
E = 320000
D = 128
H = 16
NP = 10240
NW = 32
CH = 128
CPT = 80
EP = NW * CPT * CH
RPT = NP // 16

_mesh = plsc.VectorSubcoreMesh(core_axis_name="c", subcore_axis_name="s")
_cp = pltpu.CompilerParams(use_tc_tiling_on_sc=False, needs_layout_passes=False)


def _wid():
    return lax.axis_index("s") * 2 + lax.axis_index("c")


@functools.partial(
    pl.kernel, mesh=_mesh, compiler_params=_cp,
    out_type=jax.ShapeDtypeStruct((2, NP, H), jnp.float32),
    scratch_types=[
        pltpu.VMEM((CPT, CH), jnp.int32),
        pltpu.VMEM((CH, H), jnp.float32),
        pltpu.VMEM_SHARED((NP, H), jnp.float32),
        pltpu.SemaphoreType.DMA,
    ],
)
def _sc_hist(dst2_hbm, zrows_hbm, ones_hbm, out_hbm, dsti_v, ones_v, acc_sh,
             sem):
    c = lax.axis_index("c")
    s = lax.axis_index("s")
    w = _wid()
    pltpu.sync_copy(zrows_hbm, acc_sh.at[pl.ds(s * RPT, RPT)])
    pltpu.sync_copy(ones_hbm, ones_v)
    pltpu.sync_copy(dst2_hbm.at[pl.ds(w * CPT, CPT)], dsti_v)
    plsc.subcore_barrier()

    @pl.loop(0, CPT)
    def _(i):
        pltpu.async_copy(ones_v, acc_sh.at[dsti_v.at[i]], sem, add=True)

    @pl.loop(0, CPT)
    def _(i):
        pltpu.make_async_copy(ones_v, acc_sh.at[dsti_v.at[i]], sem).wait()

    plsc.subcore_barrier()
    pltpu.sync_copy(acc_sh.at[pl.ds(s * RPT, RPT)],
                    out_hbm.at[c].at[pl.ds(s * RPT, RPT)])


@functools.partial(
    pl.kernel, mesh=_mesh, compiler_params=_cp,
    out_type=jax.ShapeDtypeStruct((2, NP, H), jnp.float32),
    scratch_types=[
        pltpu.VMEM((CPT, CH), jnp.int32),
        pltpu.VMEM((CPT, CH), jnp.int32),
        [pltpu.VMEM((CH, H), jnp.float32)] * 4,
        [pltpu.SemaphoreType.DMA] * 4,
        [pltpu.SemaphoreType.DMA] * 4,
        pltpu.VMEM_SHARED((NP, H), jnp.float32),
        pltpu.VMEM_SHARED((NP, H), jnp.float32),
    ],
)
def _sc_agg16(src2_hbm, dst2_hbm, tab_hbm, zrows_hbm, out_hbm,
              srci_v, dsti_v, rows, gsem, ssem, acc_sh, tab_sh):
    c = lax.axis_index("c")
    s = lax.axis_index("s")
    w = _wid()
    pltpu.sync_copy(zrows_hbm, acc_sh.at[pl.ds(s * RPT, RPT)])
    pltpu.sync_copy(tab_hbm.at[pl.ds(s * RPT, RPT)],
                    tab_sh.at[pl.ds(s * RPT, RPT)])
    pltpu.sync_copy(src2_hbm.at[pl.ds(w * CPT, CPT)], srci_v)
    pltpu.sync_copy(dst2_hbm.at[pl.ds(w * CPT, CPT)], dsti_v)
    plsc.subcore_barrier()

    def _gather(i, b):
        pltpu.async_copy(tab_sh.at[srci_v.at[i]], rows[b], gsem[b])

    def _wait_gather(i, b):
        pltpu.make_async_copy(tab_sh.at[srci_v.at[i]], rows[b], gsem[b]).wait()

    def _scatter(i, b):
        pltpu.async_copy(rows[b], acc_sh.at[dsti_v.at[i]], ssem[b], add=True)

    def _wait_scatter(i, b):
        pltpu.make_async_copy(rows[b], acc_sh.at[dsti_v.at[i]], ssem[b]).wait()

    _gather(0, 0)
    _gather(1, 1)

    @pl.loop(0, CPT, step=4)
    def _(g):
        for b in range(4):
            j = g + b
            _wait_gather(j, b)
            _scatter(j, b)
            i = j + 2
            bi = (b + 2) % 4

            @pl.when(i < CPT)
            def _():
                @pl.when(j >= 2)
                def _():
                    _wait_scatter(j - 2, bi)

                _gather(i, bi)

    _wait_scatter(CPT - 2, (CPT - 2) % 4)
    _wait_scatter(CPT - 1, (CPT - 1) % 4)
    plsc.subcore_barrier()
    pltpu.sync_copy(acc_sh.at[pl.ds(s * RPT, RPT)],
                    out_hbm.at[c].at[pl.ds(s * RPT, RPT)])


def _tc_mm1scale_body(hist_ref, x_ref, w1_ref, y1_ref):
    dinv = lax.rsqrt(hist_ref[0] + hist_ref[1] + 1.0)
    xw = jnp.dot(x_ref[...], w1_ref[...],
                 preferred_element_type=jnp.float32)
    y1_ref[0:N, :] = xw * dinv[0:N]
    y1_ref[N:NP, :] = jnp.zeros((NP - N, H), jnp.float32)


def _tc_layer2_body(hist_ref, agg_ref, y1_ref, b1_ref, g_ref):
    dinv = lax.rsqrt(hist_ref[0] + hist_ref[1] + 1.0)
    pre = (agg_ref[0] + agg_ref[1] + y1_ref[...]) * dinv + b1_ref[...]
    g_ref[...] = jnp.maximum(pre, 0.0) * dinv


def _tc_final_body(hist_ref, p_ref, g_ref, w2_ref, b2_ref, o_ref):
    dinv = lax.rsqrt(hist_ref[0] + hist_ref[1] + 1.0)
    t = ((p_ref[0] + p_ref[1] + g_ref[...]) * dinv)[0:N]
    z = jnp.dot(t, w2_ref[...], preferred_element_type=jnp.float32)
    o_ref[...] = z + b2_ref[...]


def kernel(x, edge_index, W1, b1, W2, b2):
    f32 = jnp.float32
    npad = EP - E
    pad_src = jnp.zeros((npad,), jnp.int32)
    pad_dst = N + (jnp.arange(npad, dtype=jnp.int32) % (NP - N))
    src2 = jnp.concatenate([edge_index[0], pad_src]).reshape(EP // CH, CH)
    dst2 = jnp.concatenate([edge_index[1], pad_dst]).reshape(EP // CH, CH)
    zrows = jnp.zeros((RPT, H), f32)
    onesr = jnp.ones((CH, H), f32)

    hist = _sc_hist(dst2, zrows, onesr)

    y1 = pl.pallas_call(
        _tc_mm1scale_body,
        out_shape=jax.ShapeDtypeStruct((NP, H), f32),
    )(hist, x, W1)

    agg = _sc_agg16(src2, dst2, y1, zrows)

    g = pl.pallas_call(
        _tc_layer2_body,
        out_shape=jax.ShapeDtypeStruct((NP, H), f32),
    )(hist, agg, y1, b1.reshape(1, H))

    p = _sc_agg16(src2, dst2, g, zrows)

    out = pl.pallas_call(
        _tc_final_body,
        out_shape=jax.ShapeDtypeStruct((N, 1), f32),
    )(hist, p, g, W2, b2.reshape(1, 1))

    return out.reshape(-1)


# speedup vs baseline: 1.4352x; 1.4352x over previous
import functools

import jax
import jax.numpy as jnp
from jax import lax
from jax.experimental import pallas as pl
from jax.experimental.pallas import tpu as pltpu
from jax.experimental.pallas import tpu_sc as plsc

N = 10000
E = 320000
D = 128
H = 16
NP = 10240
NW = 32
CH = 128
CPT = 80
EP = NW * CPT * CH
RPT = NP // 16
PK = NP * H // 128

_mesh = plsc.VectorSubcoreMesh(core_axis_name="c", subcore_axis_name="s")
_cp = pltpu.CompilerParams(use_tc_tiling_on_sc=False, needs_layout_passes=False)


def _wid():
    return lax.axis_index("s") * 2 + lax.axis_index("c")


@functools.partial(
    pl.kernel, mesh=_mesh, compiler_params=_cp,
    out_type=jax.ShapeDtypeStruct((2, NP, H), jnp.float32),
    scratch_types=[
        pltpu.VMEM((CPT, CH), jnp.int32),
        pltpu.VMEM((CH, H), jnp.float32),
        pltpu.VMEM_SHARED((NP, H), jnp.float32),
        pltpu.SemaphoreType.DMA,
    ],
)
def _sc_hist(dst2_hbm, zrows_hbm, ones_hbm, out_hbm, dsti_v, ones_v, acc_sh,
             sem):
    c = lax.axis_index("c")
    s = lax.axis_index("s")
    w = _wid()
    pltpu.sync_copy(zrows_hbm, acc_sh.at[pl.ds(s * RPT, RPT)])
    pltpu.sync_copy(ones_hbm, ones_v)
    pltpu.sync_copy(dst2_hbm.at[pl.ds(w * CPT, CPT)], dsti_v)
    plsc.subcore_barrier()

    @pl.loop(0, CPT)
    def _(i):
        pltpu.async_copy(ones_v, acc_sh.at[dsti_v.at[i]], sem, add=True)

    @pl.loop(0, CPT)
    def _(i):
        pltpu.make_async_copy(ones_v, acc_sh.at[dsti_v.at[i]], sem).wait()

    plsc.subcore_barrier()
    pltpu.sync_copy(acc_sh.at[pl.ds(s * RPT, RPT)],
                    out_hbm.at[c].at[pl.ds(s * RPT, RPT)])


@functools.partial(
    pl.kernel, mesh=_mesh, compiler_params=_cp,
    out_type=jax.ShapeDtypeStruct((2, NP, H), jnp.float32),
    scratch_types=[
        pltpu.VMEM((CPT, CH), jnp.int32),
        pltpu.VMEM((CPT, CH), jnp.int32),
        pltpu.VMEM((CH,), jnp.int32),
        [pltpu.VMEM((CH, H), jnp.float32)] * 4,
        [pltpu.SemaphoreType.DMA] * 4,
        [pltpu.SemaphoreType.DMA] * 4,
        pltpu.VMEM_SHARED((NP, H), jnp.float32),
        pltpu.VMEM_SHARED((NP, H), jnp.float32),
    ],
)
def _sc_agg16(src2_hbm, dst2_hbm, tab_hbm, zrows_hbm, out_hbm,
              srci_v, dsti_v, idn_v, rows, gsem, ssem, acc_sh, tab_sh):
    c = lax.axis_index("c")
    s = lax.axis_index("s")
    w = _wid()
    pltpu.sync_copy(zrows_hbm, acc_sh.at[pl.ds(s * RPT, RPT)])
    pltpu.sync_copy(tab_hbm.at[pl.ds(s * RPT, RPT)],
                    tab_sh.at[pl.ds(s * RPT, RPT)])
    pltpu.sync_copy(src2_hbm.at[pl.ds(w * CPT, CPT)], srci_v)
    pltpu.sync_copy(dst2_hbm.at[pl.ds(w * CPT, CPT)], dsti_v)
    plsc.subcore_barrier()

    def _gather(i, b):
        pltpu.async_copy(tab_sh.at[srci_v.at[i]], rows[b], gsem[b])

    def _wait_gather(i, b):
        pltpu.make_async_copy(tab_sh.at[srci_v.at[i]], rows[b], gsem[b]).wait()

    def _scatter(i, b):
        pltpu.async_copy(rows[b], acc_sh.at[dsti_v.at[i]], ssem[b], add=True)

    def _wait_scatter(i, b):
        pltpu.make_async_copy(rows[b], acc_sh.at[dsti_v.at[i]], ssem[b]).wait()

    _gather(0, 0)
    _gather(1, 1)

    @pl.loop(0, CPT, step=4)
    def _(g):
        for b in range(4):
            j = g + b
            _wait_gather(j, b)
            _scatter(j, b)
            i = j + 2
            bi = (b + 2) % 4

            @pl.when(i < CPT)
            def _():
                @pl.when(j >= 2)
                def _():
                    _wait_scatter(j - 2, bi)

                _gather(i, bi)

    _wait_scatter(CPT - 2, (CPT - 2) % 4)
    _wait_scatter(CPT - 1, (CPT - 1) % 4)

    @pl.when(c == 0)
    def _():
        @pl.loop(0, RPT, step=CH)
        def _(r0):
            base = s * RPT + r0

            @pl.loop(0, CH, step=16)
            def _(jj):
                idn_v[pl.ds(jj, 16)] = lax.iota(jnp.int32, 16) + base + jj

            pltpu.sync_copy(tab_sh.at[pl.ds(base, CH)], rows[0])
            pltpu.sync_copy(rows[0], acc_sh.at[idn_v], add=True)

    plsc.subcore_barrier()
    pltpu.sync_copy(acc_sh.at[pl.ds(s * RPT, RPT)],
                    out_hbm.at[c].at[pl.ds(s * RPT, RPT)])


def _tc_mm1scale_body(hist_ref, x_ref, w1_ref, y1_ref):
    dinv = lax.rsqrt(hist_ref[0] + hist_ref[1] + 1.0)
    xw = jnp.dot(x_ref[...], w1_ref[...],
                 preferred_element_type=jnp.float32)
    y1_ref[0:N, :] = xw * dinv[0:N]
    y1_ref[N:NP, :] = jnp.zeros((NP - N, H), jnp.float32)


def _tc_layer2_body(hist2_ref, agg2_ref, b1t_ref, g2_ref):
    dinv = lax.rsqrt(hist2_ref[0:PK] + hist2_ref[PK:2 * PK] + 1.0)
    pre = (agg2_ref[0:PK] + agg2_ref[PK:2 * PK]) * dinv + b1t_ref[...]
    g2_ref[...] = jnp.maximum(pre, 0.0) * dinv


def _tc_final_body(hist2_ref, p2_ref, w2b_ref, b2_ref, o_ref):
    dinv = lax.rsqrt(hist2_ref[0:PK] + hist2_ref[PK:2 * PK] + 1.0)
    t = (p2_ref[0:PK] + p2_ref[PK:2 * PK]) * dinv
    z = jnp.dot(t, w2b_ref[...], preferred_element_type=jnp.float32)
    o_ref[...] = z + b2_ref[...]


def kernel(x, edge_index, W1, b1, W2, b2):
    f32 = jnp.float32
    npad2 = (EP - E) // CH
    pad_src2 = jnp.zeros((npad2, CH), jnp.int32)
    pad_dst2 = N + (jnp.arange(npad2 * CH, dtype=jnp.int32)
                    % (NP - N)).reshape(npad2, CH)
    e2 = edge_index.reshape(2, E // CH, CH)
    src2 = jnp.concatenate([e2[0], pad_src2], axis=0)
    dst2 = jnp.concatenate([e2[1], pad_dst2], axis=0)
    zrows = jnp.zeros((RPT, H), f32)
    onesr = jnp.ones((CH, H), f32)
    b1t = jnp.tile(b1, 8).reshape(1, 128)
    w2blk = (jnp.repeat(jnp.eye(8, dtype=f32), H, axis=0)
             * jnp.tile(W2[:, 0], 8)[:, None])

    hist = _sc_hist(dst2, zrows, onesr)
    hist2 = hist.reshape(2 * PK, 128)

    y1 = pl.pallas_call(
        _tc_mm1scale_body,
        out_shape=jax.ShapeDtypeStruct((NP, H), f32),
    )(hist, x, W1)

    agg = _sc_agg16(src2, dst2, y1, zrows)

    g2 = pl.pallas_call(
        _tc_layer2_body,
        out_shape=jax.ShapeDtypeStruct((PK, 128), f32),
    )(hist2, agg.reshape(2 * PK, 128), b1t)

    p = _sc_agg16(src2, dst2, g2.reshape(NP, H), zrows)

    z = pl.pallas_call(
        _tc_final_body,
        out_shape=jax.ShapeDtypeStruct((PK, 8), f32),
    )(hist2, p.reshape(2 * PK, 128), w2blk, b2.reshape(1, 1))

    return z.reshape(NP)[:N]
